# baseline (speedup 1.0000x reference)
import functools
import jax
import jax.numpy as jnp
from jax import lax
from jax.experimental import pallas as pl
from jax.experimental.pallas import tpu as pltpu
from jax.experimental.pallas import tpu_sc as plsc

N = 5000
NP = 5120
NW = 16
C = NP // NW
NB = C // 16
KB = 8
IOU_THRESH = 0.3


def _sc_nms_body(x1h, y1h, x2h, y2h, arh, outh,
                 x1v, y1v, x2v, y2v, arv, supp, keep, klist, kbuf,
                 cntv, cbuf, shlist, shcnt):
    wid = lax.axis_index("s")
    cid = lax.axis_index("c")
    base = wid * C

    def sload(ref, idx):
        return ref[pl.ds(idx, 16)][0]

    pltpu.sync_copy(x1h, x1v.at[pl.ds(0, NP)])
    pltpu.sync_copy(y1h, y1v.at[pl.ds(0, NP)])
    pltpu.sync_copy(x2h, x2v.at[pl.ds(0, NP)])
    pltpu.sync_copy(y2h, y2v.at[pl.ds(0, NP)])
    pltpu.sync_copy(arh, arv.at[pl.ds(0, NP)])
    zf = jnp.zeros((16,), jnp.float32)
    npv = jnp.full((16,), NP, jnp.int32)
    x1v[pl.ds(NP, 16)] = zf
    y1v[pl.ds(NP, 16)] = zf
    x2v[pl.ds(NP, 16)] = zf
    y2v[pl.ds(NP, 16)] = zf
    arv[pl.ds(NP, 16)] = zf
    for b in range(NB + 1):
        supp[pl.ds(b * 16, 16)] = zf
        klist[pl.ds(b * 16, 16)] = npv
        kbuf[pl.ds(b * 16, 16)] = npv

    def batch_suppress(list_ref, j0, j1, first_blk):
        ng = (j1 - j0 + (KB - 1)) // KB

        def grp(g, _):
            jb = j0 + g * KB
            sxs = [None] * KB
            for j in range(KB):
                gi = sload(list_ref, jb + j)
                sxs[j] = (sload(x1v, gi), sload(y1v, gi),
                          sload(x2v, gi), sload(y2v, gi),
                          sload(arv, gi))

            def blk(b, _):
                off = b * 16
                tx1 = x1v[pl.ds(base + off, 16)]
                ty1 = y1v[pl.ds(base + off, 16)]
                tx2 = x2v[pl.ds(base + off, 16)]
                ty2 = y2v[pl.ds(base + off, 16)]
                tar = arv[pl.ds(base + off, 16)]
                sblk = supp[pl.ds(off, 16)]
                for j in range(KB):
                    sx1, sy1, sx2, sy2, sar = sxs[j]
                    iw = jnp.clip(
                        jnp.minimum(sx2, tx2) - jnp.maximum(sx1, tx1), 0.0)
                    ih = jnp.clip(
                        jnp.minimum(sy2, ty2) - jnp.maximum(sy1, ty1), 0.0)
                    inter = iw * ih
                    iou = inter / ((sar + tar - inter) + 1e-9)
                    sblk = jnp.where(iou > IOU_THRESH, 1.0, sblk)
                supp[pl.ds(off, 16)] = sblk
                return 0

            lax.fori_loop(first_blk, NB, blk, 0)
            return 0

        lax.fori_loop(0, ng, grp, 0)

    def round_body(w, _):
        @pl.when(wid == w)
        def _owner():
            lane = lax.broadcasted_iota(jnp.int32, (16,), 0)

            def seq_block(b, cnt):
                off = b * 16
                boff = base + off
                tx1 = x1v[pl.ds(boff, 16)]
                ty1 = y1v[pl.ds(boff, 16)]
                tx2 = x2v[pl.ds(boff, 16)]
                ty2 = y2v[pl.ds(boff, 16)]
                tar = arv[pl.ds(boff, 16)]
                sblk = supp[pl.ds(off, 16)]
                cnt0 = cnt
                hits = []
                for i in range(16):
                    iw = jnp.clip(
                        jnp.minimum(tx2[i], tx2) - jnp.maximum(tx1[i], tx1),
                        0.0)
                    ih = jnp.clip(
                        jnp.minimum(ty2[i], ty2) - jnp.maximum(ty1[i], ty1),
                        0.0)
                    inter = iw * ih
                    iou = inter / ((tar[i] + tar - inter) + 1e-9)
                    hit = jnp.logical_and(iou > IOU_THRESH, lane > i)
                    hits.append(jnp.where(hit, 1.0, 0.0))
                for i in range(16):
                    sblk = jnp.maximum(sblk, hits[i] * (1.0 - sblk[i]))
                supp[pl.ds(off, 16)] = sblk
                for i in range(16):
                    s_i = sblk[i]
                    cur = klist[pl.ds(cnt, 16)]
                    val = jnp.where(s_i == 0.0, boff + i, NP)
                    klist[pl.ds(cnt, 16)] = jnp.where(lane == 0, val, cur)
                    cnt = cnt + jnp.where(s_i == 0.0, 1, 0)
                batch_suppress(klist, cnt0, cnt, b + 1)
                return cnt

            cnt = lax.fori_loop(0, NB, seq_block, jnp.int32(0))
            for b in range(NB):
                keep[pl.ds(b * 16, 16)] = 1.0 - supp[pl.ds(b * 16, 16)]
            cntv[pl.ds(0, 16)] = jnp.full((16,), cnt, jnp.int32)

            pltpu.sync_copy(klist.at[pl.ds(0, C)], shlist.at[pl.ds(w * C, C)])
            pltpu.sync_copy(cntv.at[pl.ds(0, 16)], shcnt.at[pl.ds(w * 16, 16)])

            @pl.when(cid == 0)
            def _():
                pltpu.sync_copy(keep.at[pl.ds(0, C)], outh.at[pl.ds(base, C)])

        plsc.subcore_barrier()

        @pl.when(wid > w)
        def _applier():
            pltpu.sync_copy(shlist.at[pl.ds(w * C, C)], kbuf.at[pl.ds(0, C)])
            pltpu.sync_copy(shcnt.at[pl.ds(w * 16, 16)], cbuf.at[pl.ds(0, 16)])
            batch_suppress(kbuf, jnp.int32(0), sload(cbuf, 0), 0)

        return 0

    lax.fori_loop(0, NW, round_body, 0)


@jax.jit
def _sc_nms(x1, y1, x2, y2, ar):
    mesh = plsc.VectorSubcoreMesh(core_axis_name="c", subcore_axis_name="s")
    f = functools.partial(
        pl.kernel,
        out_type=jax.ShapeDtypeStruct((NP,), jnp.float32),
        mesh=mesh,
        scratch_types=[
            pltpu.VMEM((NP + 16,), jnp.float32),
            pltpu.VMEM((NP + 16,), jnp.float32),
            pltpu.VMEM((NP + 16,), jnp.float32),
            pltpu.VMEM((NP + 16,), jnp.float32),
            pltpu.VMEM((NP + 16,), jnp.float32),
            pltpu.VMEM((C + 16,), jnp.float32),
            pltpu.VMEM((C + 16,), jnp.float32),
            pltpu.VMEM((C + 16,), jnp.int32),
            pltpu.VMEM((C + 16,), jnp.int32),
            pltpu.VMEM((16,), jnp.int32),
            pltpu.VMEM((16,), jnp.int32),
            pltpu.VMEM_SHARED((NW * C,), jnp.int32),
            pltpu.VMEM_SHARED((NW * 16,), jnp.int32),
        ],
    )(_sc_nms_body)
    return f(x1, y1, x2, y2, ar)


def kernel(boxes, scores):
    order = jnp.argsort(-scores)
    b = jnp.take(boxes, order, axis=0)
    area = (b[:, 2] - b[:, 0]) * (b[:, 3] - b[:, 1])
    pad = jnp.zeros((NP - N,), jnp.float32)
    x1 = jnp.concatenate([b[:, 0], pad])
    y1 = jnp.concatenate([b[:, 1], pad])
    x2 = jnp.concatenate([b[:, 2], pad])
    y2 = jnp.concatenate([b[:, 3], pad])
    ar = jnp.concatenate([area, pad])
    keep_sorted = _sc_nms(x1, y1, x2, y2, ar)[:N]
    keep = jnp.zeros((N,), jnp.float32).at[order].set(keep_sorted)
    dets = jnp.concatenate([scores[:, None], boxes], axis=1)
    return dets * keep[:, None]


# speedup vs baseline: 1.0712x; 1.0712x over previous
import functools
import jax
import jax.numpy as jnp
from jax import lax
from jax.experimental import pallas as pl
from jax.experimental.pallas import tpu as pltpu
from jax.experimental.pallas import tpu_sc as plsc

N = 5000
NP = 5120
NW = 16
C = NP // NW
NB = C // 16
KB = 8
IOU_THRESH = 0.3


def _sc_nms_body(x1h, y1h, x2h, y2h, arh, outh,
                 x1v, y1v, x2v, y2v, arv, supp, keep, klist, kbuf,
                 cntv, cbuf, shlist, shcnt):
    wid = lax.axis_index("s")
    cid = lax.axis_index("c")
    base = wid * C

    def sload(ref, idx):
        return ref[pl.ds(idx, 16)][0]

    pltpu.sync_copy(x1h, x1v.at[pl.ds(0, NP)])
    pltpu.sync_copy(y1h, y1v.at[pl.ds(0, NP)])
    pltpu.sync_copy(x2h, x2v.at[pl.ds(0, NP)])
    pltpu.sync_copy(y2h, y2v.at[pl.ds(0, NP)])
    pltpu.sync_copy(arh, arv.at[pl.ds(0, NP)])
    zf = jnp.zeros((16,), jnp.float32)
    npv = jnp.full((16,), NP, jnp.int32)
    x1v[pl.ds(NP, 16)] = zf
    y1v[pl.ds(NP, 16)] = zf
    x2v[pl.ds(NP, 16)] = zf
    y2v[pl.ds(NP, 16)] = zf
    arv[pl.ds(NP, 16)] = zf
    for b in range(NB + 1):
        supp[pl.ds(b * 16, 16)] = zf
        klist[pl.ds(b * 16, 16)] = npv
        kbuf[pl.ds(b * 16, 16)] = npv

    def batch_suppress(list_ref, j0, j1, first_blk):
        ng = (j1 - j0 + (KB - 1)) // KB

        def grp(g, _):
            jb = j0 + g * KB
            sxs = [None] * KB
            for j in range(KB):
                gi = sload(list_ref, jb + j)
                sxs[j] = (sload(x1v, gi), sload(y1v, gi),
                          sload(x2v, gi), sload(y2v, gi),
                          sload(arv, gi))

            def blk(b, _):
                off = b * 16
                tx1 = x1v[pl.ds(base + off, 16)]
                ty1 = y1v[pl.ds(base + off, 16)]
                tx2 = x2v[pl.ds(base + off, 16)]
                ty2 = y2v[pl.ds(base + off, 16)]
                tar = arv[pl.ds(base + off, 16)]
                sblk = supp[pl.ds(off, 16)]
                for j in range(KB):
                    sx1, sy1, sx2, sy2, sar = sxs[j]
                    iw = jnp.clip(
                        jnp.minimum(sx2, tx2) - jnp.maximum(sx1, tx1), 0.0)
                    ih = jnp.clip(
                        jnp.minimum(sy2, ty2) - jnp.maximum(sy1, ty1), 0.0)
                    inter = iw * ih
                    iou = inter / ((sar + tar - inter) + 1e-9)
                    sblk = jnp.where(iou > IOU_THRESH, 1.0, sblk)
                supp[pl.ds(off, 16)] = sblk
                return 0

            lax.fori_loop(first_blk, NB, blk, 0)
            return 0

        lax.fori_loop(0, ng, grp, 0)

    def round_body(w, _):
        @pl.when(wid == w)
        def _owner():
            lane = lax.broadcasted_iota(jnp.int32, (16,), 0)

            def seq_block(b, cnt):
                off = b * 16
                boff = base + off
                tx1 = x1v[pl.ds(boff, 16)]
                ty1 = y1v[pl.ds(boff, 16)]
                tx2 = x2v[pl.ds(boff, 16)]
                ty2 = y2v[pl.ds(boff, 16)]
                tar = arv[pl.ds(boff, 16)]
                sblk = supp[pl.ds(off, 16)]
                cnt0 = cnt
                for i in range(16):
                    s_i = sblk[i]
                    iw = jnp.clip(
                        jnp.minimum(tx2[i], tx2) - jnp.maximum(tx1[i], tx1),
                        0.0)
                    ih = jnp.clip(
                        jnp.minimum(ty2[i], ty2) - jnp.maximum(ty1[i], ty1),
                        0.0)
                    inter = iw * ih
                    iou = inter / ((tar[i] + tar - inter) + 1e-9)
                    hit = jnp.logical_and(iou > IOU_THRESH, lane > i)
                    sblk = jnp.maximum(sblk, jnp.where(hit, 1.0 - s_i, 0.0))
                    val = jnp.where(s_i == 0.0, boff + i, NP)
                    klist[pl.ds(cnt, 16)] = jnp.where(lane == 0, val, NP)
                    cnt = cnt + jnp.where(s_i == 0.0, 1, 0)
                supp[pl.ds(off, 16)] = sblk
                batch_suppress(klist, cnt0, cnt, b + 1)
                return cnt

            cnt = lax.fori_loop(0, NB, seq_block, jnp.int32(0))
            for b in range(NB):
                keep[pl.ds(b * 16, 16)] = 1.0 - supp[pl.ds(b * 16, 16)]
            cntv[pl.ds(0, 16)] = jnp.full((16,), cnt, jnp.int32)

            pltpu.sync_copy(klist.at[pl.ds(0, C)], shlist.at[pl.ds(w * C, C)])
            pltpu.sync_copy(cntv.at[pl.ds(0, 16)], shcnt.at[pl.ds(w * 16, 16)])

            @pl.when(cid == 0)
            def _():
                pltpu.sync_copy(keep.at[pl.ds(0, C)], outh.at[pl.ds(base, C)])

        plsc.subcore_barrier()

        @pl.when(wid > w)
        def _applier():
            pltpu.sync_copy(shlist.at[pl.ds(w * C, C)], kbuf.at[pl.ds(0, C)])
            pltpu.sync_copy(shcnt.at[pl.ds(w * 16, 16)], cbuf.at[pl.ds(0, 16)])
            batch_suppress(kbuf, jnp.int32(0), sload(cbuf, 0), 0)

        return 0

    lax.fori_loop(0, NW, round_body, 0)


@jax.jit
def _sc_nms(x1, y1, x2, y2, ar):
    mesh = plsc.VectorSubcoreMesh(core_axis_name="c", subcore_axis_name="s")
    f = functools.partial(
        pl.kernel,
        out_type=jax.ShapeDtypeStruct((NP,), jnp.float32),
        mesh=mesh,
        scratch_types=[
            pltpu.VMEM((NP + 16,), jnp.float32),
            pltpu.VMEM((NP + 16,), jnp.float32),
            pltpu.VMEM((NP + 16,), jnp.float32),
            pltpu.VMEM((NP + 16,), jnp.float32),
            pltpu.VMEM((NP + 16,), jnp.float32),
            pltpu.VMEM((C + 16,), jnp.float32),
            pltpu.VMEM((C + 16,), jnp.float32),
            pltpu.VMEM((C + 16,), jnp.int32),
            pltpu.VMEM((C + 16,), jnp.int32),
            pltpu.VMEM((16,), jnp.int32),
            pltpu.VMEM((16,), jnp.int32),
            pltpu.VMEM_SHARED((NW * C,), jnp.int32),
            pltpu.VMEM_SHARED((NW * 16,), jnp.int32),
        ],
    )(_sc_nms_body)
    return f(x1, y1, x2, y2, ar)


def kernel(boxes, scores):
    order = jnp.argsort(-scores)
    b = jnp.take(boxes, order, axis=0)
    area = (b[:, 2] - b[:, 0]) * (b[:, 3] - b[:, 1])
    pad = jnp.zeros((NP - N,), jnp.float32)
    x1 = jnp.concatenate([b[:, 0], pad])
    y1 = jnp.concatenate([b[:, 1], pad])
    x2 = jnp.concatenate([b[:, 2], pad])
    y2 = jnp.concatenate([b[:, 3], pad])
    ar = jnp.concatenate([area, pad])
    keep_sorted = _sc_nms(x1, y1, x2, y2, ar)[:N]
    keep = jnp.zeros((N,), jnp.float32).at[order].set(keep_sorted)
    dets = jnp.concatenate([scores[:, None], boxes], axis=1)
    return dets * keep[:, None]


# speedup vs baseline: 1.0913x; 1.0188x over previous
import functools
import jax
import jax.numpy as jnp
from jax import lax
from jax.experimental import pallas as pl
from jax.experimental.pallas import tpu as pltpu
from jax.experimental.pallas import tpu_sc as plsc

N = 5000
NP = 5120
NW = 16
C = NP // NW
NB = C // 16
KB = 8
IOU_THRESH = 0.3


def _sc_nms_body(x1h, y1h, x2h, y2h, arh, outh,
                 x1v, y1v, x2v, y2v, arv, supp, keep, klist, kbuf,
                 cntv, cbuf, shlist, shcnt):
    wid = lax.axis_index("s")
    cid = lax.axis_index("c")
    base = wid * C

    def sload(ref, idx):
        return ref[pl.ds(idx, 16)][0]

    pltpu.sync_copy(x1h, x1v.at[pl.ds(0, NP)])
    pltpu.sync_copy(y1h, y1v.at[pl.ds(0, NP)])
    pltpu.sync_copy(x2h, x2v.at[pl.ds(0, NP)])
    pltpu.sync_copy(y2h, y2v.at[pl.ds(0, NP)])
    pltpu.sync_copy(arh, arv.at[pl.ds(0, NP)])
    zf = jnp.zeros((16,), jnp.float32)
    npv = jnp.full((16,), NP, jnp.int32)
    x1v[pl.ds(NP, 16)] = zf
    y1v[pl.ds(NP, 16)] = zf
    x2v[pl.ds(NP, 16)] = zf
    y2v[pl.ds(NP, 16)] = zf
    arv[pl.ds(NP, 16)] = zf
    for b in range(NB + 1):
        supp[pl.ds(b * 16, 16)] = zf
    for b in range(NB + 3):
        klist[pl.ds(b * 16, 16)] = npv
        kbuf[pl.ds(b * 16, 16)] = npv

    def batch_suppress(list_ref, j0, j1, first_blk):
        ng = (j1 - j0 + (KB - 1)) // KB

        def grp(g, _):
            jb = j0 + g * KB
            sxs = [None] * KB
            for j in range(KB):
                gi = sload(list_ref, jb + j)
                sxs[j] = (sload(x1v, gi), sload(y1v, gi),
                          sload(x2v, gi), sload(y2v, gi),
                          sload(arv, gi))

            def blk(b, _):
                off = b * 16
                tx1 = x1v[pl.ds(base + off, 16)]
                ty1 = y1v[pl.ds(base + off, 16)]
                tx2 = x2v[pl.ds(base + off, 16)]
                ty2 = y2v[pl.ds(base + off, 16)]
                tar = arv[pl.ds(base + off, 16)]
                sblk = supp[pl.ds(off, 16)]
                for j in range(KB):
                    sx1, sy1, sx2, sy2, sar = sxs[j]
                    iw = jnp.clip(
                        jnp.minimum(sx2, tx2) - jnp.maximum(sx1, tx1), 0.0)
                    ih = jnp.clip(
                        jnp.minimum(sy2, ty2) - jnp.maximum(sy1, ty1), 0.0)
                    inter = iw * ih
                    iou = inter / ((sar + tar - inter) + 1e-9)
                    sblk = jnp.where(iou > IOU_THRESH, 1.0, sblk)
                supp[pl.ds(off, 16)] = sblk
                return 0

            lax.fori_loop(first_blk, NB, blk, 0)
            return 0

        lax.fori_loop(0, ng, grp, 0)

    def round_body(w, _):
        @pl.when(wid == w)
        def _owner():
            lane = lax.broadcasted_iota(jnp.int32, (16,), 0)

            def seq_block(b, cnt):
                off = b * 16
                boff = base + off
                tx1 = x1v[pl.ds(boff, 16)]
                ty1 = y1v[pl.ds(boff, 16)]
                tx2 = x2v[pl.ds(boff, 16)]
                ty2 = y2v[pl.ds(boff, 16)]
                tar = arv[pl.ds(boff, 16)]
                sblk = supp[pl.ds(off, 16)]
                cnt0 = cnt
                dn = lax.GatherDimensionNumbers(
                    offset_dims=(), collapsed_slice_dims=(0,),
                    start_index_map=(0,))
                for i in range(16):
                    iw = jnp.clip(
                        jnp.minimum(tx2[i], tx2) - jnp.maximum(tx1[i], tx1),
                        0.0)
                    ih = jnp.clip(
                        jnp.minimum(ty2[i], ty2) - jnp.maximum(ty1[i], ty1),
                        0.0)
                    inter = iw * ih
                    iou = inter / ((tar[i] + tar - inter) + 1e-9)
                    hit = jnp.logical_and(iou > IOU_THRESH, lane > i)
                    sbc = lax.gather(
                        sblk, jnp.full((16, 1), i, jnp.int32), dn,
                        slice_sizes=(1,),
                        mode=lax.GatherScatterMode.PROMISE_IN_BOUNDS)
                    s_i = sblk[i]
                    sblk = jnp.maximum(sblk, jnp.where(hit, 1.0 - sbc, 0.0))
                    val = jnp.where(s_i == 0.0, boff + i, NP)
                    klist[pl.ds(cnt, 16)] = jnp.where(lane == 0, val, NP)
                    cnt = cnt + jnp.where(s_i == 0.0, 1, 0)
                supp[pl.ds(off, 16)] = sblk
                batch_suppress(klist, cnt0, cnt, b + 1)
                return cnt

            cnt = lax.fori_loop(0, NB, seq_block, jnp.int32(0))
            for b in range(NB):
                keep[pl.ds(b * 16, 16)] = 1.0 - supp[pl.ds(b * 16, 16)]
            cntv[pl.ds(0, 16)] = jnp.full((16,), cnt, jnp.int32)

            pltpu.sync_copy(klist.at[pl.ds(0, C)], shlist.at[pl.ds(w * C, C)])
            pltpu.sync_copy(cntv.at[pl.ds(0, 16)], shcnt.at[pl.ds(w * 16, 16)])

            @pl.when(cid == 0)
            def _():
                pltpu.sync_copy(keep.at[pl.ds(0, C)], outh.at[pl.ds(base, C)])

        plsc.subcore_barrier()

        @pl.when(wid > w)
        def _applier():
            pltpu.sync_copy(shlist.at[pl.ds(w * C, C)], kbuf.at[pl.ds(0, C)])
            pltpu.sync_copy(shcnt.at[pl.ds(w * 16, 16)], cbuf.at[pl.ds(0, 16)])
            batch_suppress(kbuf, jnp.int32(0), sload(cbuf, 0), 0)

        return 0

    lax.fori_loop(0, NW, round_body, 0)


@jax.jit
def _sc_nms(x1, y1, x2, y2, ar):
    mesh = plsc.VectorSubcoreMesh(core_axis_name="c", subcore_axis_name="s")
    f = functools.partial(
        pl.kernel,
        out_type=jax.ShapeDtypeStruct((NP,), jnp.float32),
        mesh=mesh,
        scratch_types=[
            pltpu.VMEM((NP + 16,), jnp.float32),
            pltpu.VMEM((NP + 16,), jnp.float32),
            pltpu.VMEM((NP + 16,), jnp.float32),
            pltpu.VMEM((NP + 16,), jnp.float32),
            pltpu.VMEM((NP + 16,), jnp.float32),
            pltpu.VMEM((C + 16,), jnp.float32),
            pltpu.VMEM((C + 16,), jnp.float32),
            pltpu.VMEM((C + 48,), jnp.int32),
            pltpu.VMEM((C + 48,), jnp.int32),
            pltpu.VMEM((16,), jnp.int32),
            pltpu.VMEM((16,), jnp.int32),
            pltpu.VMEM_SHARED((NW * C,), jnp.int32),
            pltpu.VMEM_SHARED((NW * 16,), jnp.int32),
        ],
    )(_sc_nms_body)
    return f(x1, y1, x2, y2, ar)


def kernel(boxes, scores):
    order = jnp.argsort(-scores)
    b = jnp.take(boxes, order, axis=0)
    area = (b[:, 2] - b[:, 0]) * (b[:, 3] - b[:, 1])
    pad = jnp.zeros((NP - N,), jnp.float32)
    x1 = jnp.concatenate([b[:, 0], pad])
    y1 = jnp.concatenate([b[:, 1], pad])
    x2 = jnp.concatenate([b[:, 2], pad])
    y2 = jnp.concatenate([b[:, 3], pad])
    ar = jnp.concatenate([area, pad])
    keep_sorted = _sc_nms(x1, y1, x2, y2, ar)[:N]
    keep = jnp.zeros((N,), jnp.float32).at[order].set(keep_sorted)
    dets = jnp.concatenate([scores[:, None], boxes], axis=1)
    return dets * keep[:, None]
